# baseline (speedup 1.0000x reference)
import functools

import jax
import jax.numpy as jnp
from jax.experimental import pallas as pl
from jax.experimental.pallas import tpu as pltpu

N = 4096
DIM = 128
KNN = 10
MARGIN1 = 1.3
MARGIN2 = 0.5
ROWS = 256
GRID = N // ROWS
BIG = 1e30
TOPL = 3
NTILE = N // 128


def _edb_kernel(xb_ref, lb_ref, xa_ref, la_ref, out_ref, u_ref):
    i = pl.program_id(0)
    xb = xb_ref[...]
    xa = xa_ref[...]
    lb = lb_ref[...]
    la = la_ref[...]

    dn = (((1,), (1,)), ((), ()))
    g2 = jax.lax.dot_general(xb * (-2.0), xa, dn,
                             preferred_element_type=jnp.float32)
    ones_r = jnp.ones((1, DIM), jnp.float32)
    sq_b = jax.lax.dot_general(xb * xb, ones_r, dn,
                               preferred_element_type=jnp.float32)
    sq_a = jax.lax.dot_general(ones_r, xa * xa, dn,
                               preferred_element_type=jnp.float32)
    dsq = jnp.maximum(sq_b + sq_a + g2, 1e-12)
    mask = lb == la

    bits = jax.lax.bitcast_convert_type(dsq, jnp.int32)
    u_ref[...] = jax.lax.bitcast_convert_type(
        (bits & jnp.int32(-2)) | mask.astype(jnp.int32), jnp.float32)

    def tile(k):
        return u_ref[:, k * 128:(k + 1) * 128]

    prev = functools.reduce(jnp.minimum,
                            [tile(k) for k in range(NTILE)])
    cands = [prev]
    for _ in range(TOPL - 1):
        cur = functools.reduce(
            jnp.minimum,
            [jnp.where(tile(k) > prev, tile(k), BIG) for k in range(NTILE)])
        cands.append(cur)
        prev = cur

    c = jnp.concatenate(cands, axis=1)
    v = jnp.min(c, axis=1, keepdims=True)
    uvals = [v]
    for _ in range(KNN - 1):
        v = jnp.min(jnp.where(c > v, c, BIG), axis=1, keepdims=True)
        uvals.append(v)

    d_all = jnp.sqrt(u_ref[...])
    s_same = jnp.sum(jnp.where(mask, d_all, 0.0), axis=1, keepdims=True)
    c_same = jnp.sum(mask.astype(jnp.float32), axis=1, keepdims=True)

    us = jnp.concatenate(uvals, axis=1)
    mf = (jax.lax.bitcast_convert_type(us, jnp.int32)
          & jnp.int32(1)).astype(jnp.float32)
    dv = jnp.sqrt(us)
    border = dv[:, KNN - 1:KNN]
    ae_cnt = jnp.sum(mf, axis=1, keepdims=True)
    same_topk_sum = jnp.sum(mf * dv, axis=1, keepdims=True)
    s_topk = jnp.sum(dv, axis=1, keepdims=True)
    ae_sum = jnp.sum(mf * jnp.maximum(MARGIN2 - dv, 0.0), axis=1,
                     keepdims=True)
    an_cnt = KNN - ae_cnt
    an_sum = (border + MARGIN1) * an_cnt - (s_topk - same_topk_sum)

    ap_cnt = c_same - ae_cnt
    ap_sum = s_same - same_topk_sum - border * ap_cnt
    ap_row = jnp.where(ap_cnt > 0, ap_sum / jnp.maximum(ap_cnt, 1.0), 0.0)
    an_row = jnp.where(an_cnt > 0, an_sum / jnp.maximum(an_cnt, 1.0), 0.0)
    ae_row = jnp.where(ae_cnt > 0, ae_sum / jnp.maximum(ae_cnt, 1.0), 0.0)

    part = jnp.concatenate([ap_row, an_row, ae_row], axis=1)
    part = jnp.sum(part, axis=0, keepdims=True)

    @pl.when(i == 0)
    def _init():
        out_ref[...] = jnp.zeros_like(out_ref)

    out_ref[...] += part

    @pl.when(i == GRID - 1)
    def _final():
        out_ref[...] = out_ref[...] * (1.0 / N)


def kernel(inputs, labels):
    lab = labels.astype(jnp.int32)
    out = pl.pallas_call(
        _edb_kernel,
        grid=(GRID,),
        in_specs=[
            pl.BlockSpec((ROWS, DIM), lambda i: (i, 0)),
            pl.BlockSpec((ROWS, 1), lambda i: (i, 0)),
            pl.BlockSpec((N, DIM), lambda i: (0, 0)),
            pl.BlockSpec((1, N), lambda i: (0, 0)),
        ],
        out_specs=pl.BlockSpec((1, 3), lambda i: (0, 0)),
        out_shape=jax.ShapeDtypeStruct((1, 3), jnp.float32),
        scratch_shapes=[pltpu.VMEM((ROWS, N), jnp.float32)],
    )(inputs, lab.reshape(N, 1), inputs, lab.reshape(1, N))
    return (out[0, 0], out[0, 1], out[0, 2])


# speedup vs baseline: 1.0956x; 1.0956x over previous
import functools

import jax
import jax.numpy as jnp
from jax.experimental import pallas as pl

N = 4096
DIM = 128
KNN = 10
MARGIN1 = 1.3
MARGIN2 = 0.5
ROWS = 256
GRID = N // ROWS
BIG = 1e30
TOPL = 3
NTILE = N // 128


def _edb_kernel(xb_ref, lb_ref, xa_ref, la_ref, out_ref):
    i = pl.program_id(0)
    xb = xb_ref[...]
    xa = xa_ref[...]
    lb = lb_ref[...]
    la = la_ref[...]

    g = jax.lax.dot_general(xb, xa, (((1,), (1,)), ((), ())),
                            preferred_element_type=jnp.float32)
    sq_b = jnp.sum(xb * xb, axis=1, keepdims=True)
    sq_a = jnp.sum(xa * xa, axis=1)[None, :]
    dsq = jnp.maximum(sq_b + sq_a - 2.0 * g, 1e-12)
    mask = lb == la

    bits = jax.lax.bitcast_convert_type(dsq, jnp.int32)
    u = jax.lax.bitcast_convert_type(
        (bits & jnp.int32(-2)) | mask.astype(jnp.int32), jnp.float32)

    tiles = [u[:, k * 128:(k + 1) * 128] for k in range(NTILE)]
    prev = functools.reduce(jnp.minimum, tiles)
    cands = [prev]
    for _ in range(TOPL - 1):
        cur = functools.reduce(
            jnp.minimum, [jnp.where(t > prev, t, BIG) for t in tiles])
        cands.append(cur)
        prev = cur

    c = jnp.concatenate(cands, axis=1)
    v = jnp.min(c, axis=1, keepdims=True)
    uvals = [v]
    for _ in range(KNN - 1):
        v = jnp.min(jnp.where(c > v, c, BIG), axis=1, keepdims=True)
        uvals.append(v)

    d_all = jnp.sqrt(u)
    s_same = jnp.sum(jnp.where(mask, d_all, 0.0), axis=1, keepdims=True)
    c_same = jnp.sum(mask.astype(jnp.float32), axis=1, keepdims=True)

    us = jnp.concatenate(uvals, axis=1)
    mf = (jax.lax.bitcast_convert_type(us, jnp.int32)
          & jnp.int32(1)).astype(jnp.float32)
    dv = jnp.sqrt(us)
    border = dv[:, KNN - 1:KNN]
    ae_cnt = jnp.sum(mf, axis=1, keepdims=True)
    same_topk_sum = jnp.sum(mf * dv, axis=1, keepdims=True)
    s_topk = jnp.sum(dv, axis=1, keepdims=True)
    ae_sum = jnp.sum(mf * jnp.maximum(MARGIN2 - dv, 0.0), axis=1,
                     keepdims=True)
    an_cnt = KNN - ae_cnt
    an_sum = (border + MARGIN1) * an_cnt - (s_topk - same_topk_sum)

    ap_cnt = c_same - ae_cnt
    ap_sum = s_same - same_topk_sum - border * ap_cnt
    ap_row = jnp.where(ap_cnt > 0, ap_sum / jnp.maximum(ap_cnt, 1.0), 0.0)
    an_row = jnp.where(an_cnt > 0, an_sum / jnp.maximum(an_cnt, 1.0), 0.0)
    ae_row = jnp.where(ae_cnt > 0, ae_sum / jnp.maximum(ae_cnt, 1.0), 0.0)

    part = jnp.concatenate([ap_row, an_row, ae_row], axis=1)
    part = jnp.sum(part, axis=0, keepdims=True)

    @pl.when(i == 0)
    def _init():
        out_ref[...] = jnp.zeros_like(out_ref)

    out_ref[...] += part

    @pl.when(i == GRID - 1)
    def _final():
        out_ref[...] = out_ref[...] * (1.0 / N)


def kernel(inputs, labels):
    lab = labels.astype(jnp.int32)
    out = pl.pallas_call(
        _edb_kernel,
        grid=(GRID,),
        in_specs=[
            pl.BlockSpec((ROWS, DIM), lambda i: (i, 0)),
            pl.BlockSpec((ROWS, 1), lambda i: (i, 0)),
            pl.BlockSpec((N, DIM), lambda i: (0, 0)),
            pl.BlockSpec((1, N), lambda i: (0, 0)),
        ],
        out_specs=pl.BlockSpec((1, 3), lambda i: (0, 0)),
        out_shape=jax.ShapeDtypeStruct((1, 3), jnp.float32),
    )(inputs, lab.reshape(N, 1), inputs, lab.reshape(1, N))
    return (out[0, 0], out[0, 1], out[0, 2])


# speedup vs baseline: 1.1685x; 1.0665x over previous
import functools

import jax
import jax.numpy as jnp
from jax.experimental import pallas as pl

N = 4096
DIM = 128
KNN = 10
MARGIN1 = 1.3
MARGIN2 = 0.5
ROWS = 512
GRID = N // ROWS
BIG = 1e30
TOPL = 3
NTILE = N // 128


def _edb_kernel(xb_ref, lb_ref, xa_ref, la_ref, out_ref):
    i = pl.program_id(0)
    xb = xb_ref[...]
    xa = xa_ref[...]
    lb = lb_ref[...]
    la = la_ref[...]

    g = jax.lax.dot_general(xb, xa, (((1,), (1,)), ((), ())),
                            preferred_element_type=jnp.float32)
    sq_b = jnp.sum(xb * xb, axis=1, keepdims=True)
    sq_a = jnp.sum(xa * xa, axis=1)[None, :]
    dsq = jnp.maximum(sq_b + sq_a - 2.0 * g, 1e-12)
    mask = lb == la

    bits = jax.lax.bitcast_convert_type(dsq, jnp.int32)
    u = jax.lax.bitcast_convert_type(
        (bits & jnp.int32(-2)) | mask.astype(jnp.int32), jnp.float32)

    tiles = [u[:, k * 128:(k + 1) * 128] for k in range(NTILE)]
    prev = functools.reduce(jnp.minimum, tiles)
    cands = [prev]
    for _ in range(TOPL - 1):
        cur = functools.reduce(
            jnp.minimum, [jnp.where(t > prev, t, BIG) for t in tiles])
        cands.append(cur)
        prev = cur

    c = jnp.concatenate(cands, axis=1)
    v = jnp.min(c, axis=1, keepdims=True)
    uvals = [v]
    for _ in range(KNN - 1):
        v = jnp.min(jnp.where(c > v, c, BIG), axis=1, keepdims=True)
        uvals.append(v)

    d_all = jnp.sqrt(u)
    s_same = jnp.sum(jnp.where(mask, d_all, 0.0), axis=1, keepdims=True)
    c_same = jnp.sum(mask.astype(jnp.float32), axis=1, keepdims=True)

    us = jnp.concatenate(uvals, axis=1)
    mf = (jax.lax.bitcast_convert_type(us, jnp.int32)
          & jnp.int32(1)).astype(jnp.float32)
    dv = jnp.sqrt(us)
    border = dv[:, KNN - 1:KNN]
    ae_cnt = jnp.sum(mf, axis=1, keepdims=True)
    same_topk_sum = jnp.sum(mf * dv, axis=1, keepdims=True)
    s_topk = jnp.sum(dv, axis=1, keepdims=True)
    ae_sum = jnp.sum(mf * jnp.maximum(MARGIN2 - dv, 0.0), axis=1,
                     keepdims=True)
    an_cnt = KNN - ae_cnt
    an_sum = (border + MARGIN1) * an_cnt - (s_topk - same_topk_sum)

    ap_cnt = c_same - ae_cnt
    ap_sum = s_same - same_topk_sum - border * ap_cnt
    ap_row = jnp.where(ap_cnt > 0, ap_sum / jnp.maximum(ap_cnt, 1.0), 0.0)
    an_row = jnp.where(an_cnt > 0, an_sum / jnp.maximum(an_cnt, 1.0), 0.0)
    ae_row = jnp.where(ae_cnt > 0, ae_sum / jnp.maximum(ae_cnt, 1.0), 0.0)

    part = jnp.concatenate([ap_row, an_row, ae_row], axis=1)
    part = jnp.sum(part, axis=0, keepdims=True)

    @pl.when(i == 0)
    def _init():
        out_ref[...] = jnp.zeros_like(out_ref)

    out_ref[...] += part

    @pl.when(i == GRID - 1)
    def _final():
        out_ref[...] = out_ref[...] * (1.0 / N)


def kernel(inputs, labels):
    lab = labels.astype(jnp.int32)
    out = pl.pallas_call(
        _edb_kernel,
        grid=(GRID,),
        in_specs=[
            pl.BlockSpec((ROWS, DIM), lambda i: (i, 0)),
            pl.BlockSpec((ROWS, 1), lambda i: (i, 0)),
            pl.BlockSpec((N, DIM), lambda i: (0, 0)),
            pl.BlockSpec((1, N), lambda i: (0, 0)),
        ],
        out_specs=pl.BlockSpec((1, 3), lambda i: (0, 0)),
        out_shape=jax.ShapeDtypeStruct((1, 3), jnp.float32),
    )(inputs, lab.reshape(N, 1), inputs, lab.reshape(1, N))
    return (out[0, 0], out[0, 1], out[0, 2])


# speedup vs baseline: 1.2500x; 1.0698x over previous
import functools

import jax
import jax.numpy as jnp
from jax.experimental import pallas as pl

N = 4096
DIM = 128
KNN = 10
MARGIN1 = 1.3
MARGIN2 = 0.5
ROWS = 512
GRID = N // ROWS
BIG = 1e30
TOPL = 3
NTILE = N // 128


def _edb_kernel(xb_ref, lb_ref, xa_ref, la_ref, out_ref):
    i = pl.program_id(0)
    xb = xb_ref[...]
    xa = xa_ref[...]
    lb = lb_ref[...]
    la = la_ref[...]

    dn = (((1,), (1,)), ((), ()))
    g2 = jax.lax.dot_general(xb * (-2.0), xa, dn,
                             preferred_element_type=jnp.float32)
    sq_b = jnp.sum(xb * xb, axis=1, keepdims=True)
    sq_a = jnp.sum(xa * xa, axis=1)[None, :]
    dsq = jnp.maximum(sq_b + sq_a + g2, 1e-12)
    mask = lb == la

    bits = jax.lax.bitcast_convert_type(dsq, jnp.int32)
    u = jax.lax.bitcast_convert_type(
        (bits & jnp.int32(-2)) | mask.astype(jnp.int32), jnp.float32)

    tiles = [u[:, k * 128:(k + 1) * 128] for k in range(NTILE)]
    prev = functools.reduce(jnp.minimum, tiles)
    cands = [prev]
    for _ in range(TOPL - 1):
        cur = functools.reduce(
            jnp.minimum, [jnp.where(t > prev, t, BIG) for t in tiles])
        cands.append(cur)
        prev = cur

    c = jnp.concatenate(cands, axis=1)
    v = jnp.min(c, axis=1, keepdims=True)
    uvals = [v]
    for _ in range(KNN - 1):
        v = jnp.min(jnp.where(c > v, c, BIG), axis=1, keepdims=True)
        uvals.append(v)

    r = jax.lax.rsqrt(u)
    t = u * r
    d_all = t * (1.5 - 0.5 * (t * r))
    ones_n = jnp.ones((1, N), jnp.float32)
    s_same = jax.lax.dot_general(jnp.where(mask, d_all, 0.0), ones_n, dn,
                                 preferred_element_type=jnp.float32)
    c_same = jax.lax.dot_general(mask.astype(jnp.float32), ones_n, dn,
                                 preferred_element_type=jnp.float32)

    us = jnp.concatenate(uvals, axis=1)
    mf = (jax.lax.bitcast_convert_type(us, jnp.int32)
          & jnp.int32(1)).astype(jnp.float32)
    dv = jnp.sqrt(us)
    border = dv[:, KNN - 1:KNN]
    ae_cnt = jnp.sum(mf, axis=1, keepdims=True)
    same_topk_sum = jnp.sum(mf * dv, axis=1, keepdims=True)
    s_topk = jnp.sum(dv, axis=1, keepdims=True)
    ae_sum = jnp.sum(mf * jnp.maximum(MARGIN2 - dv, 0.0), axis=1,
                     keepdims=True)
    an_cnt = KNN - ae_cnt
    an_sum = (border + MARGIN1) * an_cnt - (s_topk - same_topk_sum)

    ap_cnt = c_same - ae_cnt
    ap_sum = s_same - same_topk_sum - border * ap_cnt
    ap_row = jnp.where(ap_cnt > 0, ap_sum / jnp.maximum(ap_cnt, 1.0), 0.0)
    an_row = jnp.where(an_cnt > 0, an_sum / jnp.maximum(an_cnt, 1.0), 0.0)
    ae_row = jnp.where(ae_cnt > 0, ae_sum / jnp.maximum(ae_cnt, 1.0), 0.0)

    part = jnp.concatenate([ap_row, an_row, ae_row], axis=1)
    part = jnp.sum(part, axis=0, keepdims=True)

    @pl.when(i == 0)
    def _init():
        out_ref[...] = jnp.zeros_like(out_ref)

    out_ref[...] += part

    @pl.when(i == GRID - 1)
    def _final():
        out_ref[...] = out_ref[...] * (1.0 / N)


def kernel(inputs, labels):
    lab = labels.astype(jnp.int32)
    out = pl.pallas_call(
        _edb_kernel,
        grid=(GRID,),
        in_specs=[
            pl.BlockSpec((ROWS, DIM), lambda i: (i, 0)),
            pl.BlockSpec((ROWS, 1), lambda i: (i, 0)),
            pl.BlockSpec((N, DIM), lambda i: (0, 0)),
            pl.BlockSpec((1, N), lambda i: (0, 0)),
        ],
        out_specs=pl.BlockSpec((1, 3), lambda i: (0, 0)),
        out_shape=jax.ShapeDtypeStruct((1, 3), jnp.float32),
    )(inputs, lab.reshape(N, 1), inputs, lab.reshape(1, N))
    return (out[0, 0], out[0, 1], out[0, 2])


# speedup vs baseline: 1.2979x; 1.0383x over previous
import functools

import jax
import jax.numpy as jnp
from jax.experimental import pallas as pl

N = 4096
DIM = 128
KNN = 10
MARGIN1 = 1.3
MARGIN2 = 0.5
ROWS = 1024
GRID = N // ROWS
BIG = 1e30
TOPL = 3
NTILE = N // 128


def _edb_kernel(xb_ref, lb_ref, xa_ref, la_ref, out_ref):
    i = pl.program_id(0)
    xb = xb_ref[...]
    xa = xa_ref[...]
    lb = lb_ref[...]
    la = la_ref[...]

    dn = (((1,), (1,)), ((), ()))
    g2 = jax.lax.dot_general(xb * (-2.0), xa, dn,
                             preferred_element_type=jnp.float32)
    sq_b = jnp.sum(xb * xb, axis=1, keepdims=True)
    sq_a = jnp.sum(xa * xa, axis=1)[None, :]
    dsq = jnp.maximum(sq_b + sq_a + g2, 1e-12)
    mask = lb == la

    bits = jax.lax.bitcast_convert_type(dsq, jnp.int32)
    u = jax.lax.bitcast_convert_type(
        (bits & jnp.int32(-2)) | mask.astype(jnp.int32), jnp.float32)

    tiles = [u[:, k * 128:(k + 1) * 128] for k in range(NTILE)]
    prev = functools.reduce(jnp.minimum, tiles)
    cands = [prev]
    for _ in range(TOPL - 1):
        cur = functools.reduce(
            jnp.minimum, [jnp.where(t > prev, t, BIG) for t in tiles])
        cands.append(cur)
        prev = cur

    c = jnp.concatenate(cands, axis=1)
    v = jnp.min(c, axis=1, keepdims=True)
    uvals = [v]
    for _ in range(KNN - 1):
        v = jnp.min(jnp.where(c > v, c, BIG), axis=1, keepdims=True)
        uvals.append(v)

    r = jax.lax.rsqrt(u)
    t = u * r
    d_all = t * (1.5 - 0.5 * (t * r))
    ones_n = jnp.ones((1, N), jnp.float32)
    s_same = jax.lax.dot_general(jnp.where(mask, d_all, 0.0), ones_n, dn,
                                 preferred_element_type=jnp.float32)
    c_same = jax.lax.dot_general(mask.astype(jnp.float32), ones_n, dn,
                                 preferred_element_type=jnp.float32)

    us = jnp.concatenate(uvals, axis=1)
    mf = (jax.lax.bitcast_convert_type(us, jnp.int32)
          & jnp.int32(1)).astype(jnp.float32)
    dv = jnp.sqrt(us)
    border = dv[:, KNN - 1:KNN]
    ae_cnt = jnp.sum(mf, axis=1, keepdims=True)
    same_topk_sum = jnp.sum(mf * dv, axis=1, keepdims=True)
    s_topk = jnp.sum(dv, axis=1, keepdims=True)
    ae_sum = jnp.sum(mf * jnp.maximum(MARGIN2 - dv, 0.0), axis=1,
                     keepdims=True)
    an_cnt = KNN - ae_cnt
    an_sum = (border + MARGIN1) * an_cnt - (s_topk - same_topk_sum)

    ap_cnt = c_same - ae_cnt
    ap_sum = s_same - same_topk_sum - border * ap_cnt
    ap_row = jnp.where(ap_cnt > 0, ap_sum / jnp.maximum(ap_cnt, 1.0), 0.0)
    an_row = jnp.where(an_cnt > 0, an_sum / jnp.maximum(an_cnt, 1.0), 0.0)
    ae_row = jnp.where(ae_cnt > 0, ae_sum / jnp.maximum(ae_cnt, 1.0), 0.0)

    part = jnp.concatenate([ap_row, an_row, ae_row], axis=1)
    part = jnp.sum(part, axis=0, keepdims=True)

    @pl.when(i == 0)
    def _init():
        out_ref[...] = jnp.zeros_like(out_ref)

    out_ref[...] += part

    @pl.when(i == GRID - 1)
    def _final():
        out_ref[...] = out_ref[...] * (1.0 / N)


def kernel(inputs, labels):
    lab = labels.astype(jnp.int32)
    out = pl.pallas_call(
        _edb_kernel,
        grid=(GRID,),
        in_specs=[
            pl.BlockSpec((ROWS, DIM), lambda i: (i, 0)),
            pl.BlockSpec((ROWS, 1), lambda i: (i, 0)),
            pl.BlockSpec((N, DIM), lambda i: (0, 0)),
            pl.BlockSpec((1, N), lambda i: (0, 0)),
        ],
        out_specs=pl.BlockSpec((1, 3), lambda i: (0, 0)),
        out_shape=jax.ShapeDtypeStruct((1, 3), jnp.float32),
    )(inputs, lab.reshape(N, 1), inputs, lab.reshape(1, N))
    return (out[0, 0], out[0, 1], out[0, 2])
